# baseline (speedup 1.0000x reference)
import functools

import jax
import jax.numpy as jnp
from jax import lax
from jax.experimental import pallas as pl
from jax.experimental.pallas import tpu as pltpu
from jax.experimental.pallas import tpu_sc as plsc

N = 10000
E = 320000
D = 128
H = 128
CH = 128
NCH = E // CH
NS = 16
NC = 2
NW = NC * NS
BN = 1000
GRID = N // BN
NP = 10240
NCHP = 2560
EP = NCHP * CH
NPAD = 16
NB = 2
SEGA = 32
SEGT = 16

_F32 = jnp.float32


def _sc_mesh():
    return plsc.VectorSubcoreMesh(core_axis_name="c", subcore_axis_name="s")


@functools.partial(
    pl.kernel,
    mesh=_sc_mesh(),
    out_type=(jax.ShapeDtypeStruct((NP,), _F32),
              jax.ShapeDtypeStruct((NP,), _F32)),
    scratch_types=[
        pltpu.VMEM((NCHP // NW, CH), jnp.int32),
        pltpu.VMEM((CH,), _F32),
        pltpu.VMEM_SHARED((NP,), _F32),
        pltpu.SemaphoreType.DMA,
        pltpu.SemaphoreType.DMA,
        pltpu.SemaphoreType.DMA,
        pltpu.SemaphoreType.DMA,
    ],
)
def _deg_sc(dst_hbm, zrow_hbm, out0_hbm, out1_hbm, didx_all, ones_v, acc_sh,
            *sems):
    c = lax.axis_index("c")
    s = lax.axis_index("s")
    w = c * NS + s
    kpw = NCHP // NW
    one = jnp.full((16,), 1.0, _F32)
    for i in range(CH // 16):
        ones_v[pl.ds(i * 16, 16)] = one
    pltpu.sync_copy(dst_hbm.at[pl.ds(w * kpw, kpw)], didx_all)

    @pl.when(s == 0)
    def _():
        pltpu.sync_copy(zrow_hbm, acc_sh)

    plsc.subcore_barrier()

    def scat(j, i):
        return pltpu.async_copy(ones_v, acc_sh.at[didx_all.at[j]], sems[i],
                                add=True)

    for i in range(NB):
        scat(i, i)

    def body(k4, carry):
        for i in range(NB):
            j = NB * k4 + i
            pltpu.make_async_copy(ones_v, acc_sh.at[didx_all.at[j]],
                                  sems[i]).wait()
            scat(j + NB, i)
        return carry

    lax.fori_loop(0, kpw // NB - 1, body, 0)
    for i in range(NB):
        j = kpw - NB + i
        pltpu.make_async_copy(ones_v, acc_sh.at[didx_all.at[j]],
                              sems[i]).wait()
    plsc.subcore_barrier()
    _writeback_1d(acc_sh, out0_hbm, out1_hbm, c, s)


def _writeback_1d(acc_sh, out0_hbm, out1_hbm, c, s):
    def wb(out_hbm):
        pltpu.sync_copy(acc_sh.at[pl.ds(s * 640, 640)],
                        out_hbm.at[pl.ds(s * 640, 640)])

    @pl.when(c == 0)
    def _():
        wb(out0_hbm)

    @pl.when(c == 1)
    def _():
        wb(out1_hbm)


@functools.partial(
    pl.kernel,
    mesh=_sc_mesh(),
    out_type=jax.ShapeDtypeStruct((NC, N, H), _F32),
    scratch_types=[
        pltpu.VMEM((SEGT, CH), jnp.int32),
        pltpu.VMEM((SEGT, CH), jnp.int32),
        [pltpu.VMEM((CH, H), _F32)] * NB,
        pltpu.VMEM_SHARED((N + NPAD, H), _F32),
        [pltpu.SemaphoreType.DMA] * NB,
        [pltpu.SemaphoreType.DMA] * NB,
    ],
)
def _agg_sc(xd_hbm, src_hbm, dst_hbm, zeros_hbm, out_hbm,
            sidx_seg, didx_seg, rows, acc_sh, gsem, ssem):
    c = lax.axis_index("c")
    s = lax.axis_index("s")
    w = c * NS + s
    kpw = NCHP // NW

    @pl.when(s == 0)
    def _():
        pltpu.sync_copy(zeros_hbm, acc_sh)

    plsc.subcore_barrier()
    _seg_pipeline(xd_hbm, acc_sh, src_hbm, dst_hbm, sidx_seg, didx_seg,
                  rows, gsem, ssem, w * kpw, kpw // SEGT, SEGT)
    plsc.subcore_barrier()
    _writeback_rows(acc_sh, out_hbm, c, s)


def _seg_pipeline(table, acc_sh, src_hbm, dst_hbm, sidx_seg, didx_seg,
                  rows, gsem, ssem, tile_base, nseg, seg):
    def gat(m, i):
        pltpu.async_copy(table.at[sidx_seg.at[m]], rows[i], gsem[i])

    def wait_g(m, i):
        pltpu.make_async_copy(table.at[sidx_seg.at[m]], rows[i],
                              gsem[i]).wait()

    def scat(m, i):
        pltpu.async_copy(rows[i], acc_sh.at[didx_seg.at[m]], ssem[i],
                         add=True)

    def wait_s(m, i):
        pltpu.make_async_copy(rows[i], acc_sh.at[didx_seg.at[m]],
                              ssem[i]).wait()

    def seg_body(g, carry):
        segbase = tile_base + g * seg
        pltpu.sync_copy(src_hbm.at[pl.ds(segbase, seg)], sidx_seg)
        pltpu.sync_copy(dst_hbm.at[pl.ds(segbase, seg)], didx_seg)
        for i in range(NB):
            gat(i, i)

        def pair(m2, carry2):
            for i in range(NB):
                m = NB * m2 + i
                wait_g(m, i)
                scat(m, i)
            for i in range(NB):
                m = NB * m2 + i
                wait_s(m, i)
                gat(m + NB, i)
            return carry2

        lax.fori_loop(0, seg // NB - 1, pair, 0)
        for i in range(NB):
            m = seg - NB + i
            wait_g(m, i)
            scat(m, i)
        for i in range(NB):
            m = seg - NB + i
            wait_s(m, i)
        return carry

    lax.fori_loop(0, nseg, seg_body, 0)


def _writeback_rows(acc_sh, out_hbm, c, s):
    @pl.when(s < NS - 1)
    def _():
        pltpu.sync_copy(acc_sh.at[pl.ds(s * 624, 624)],
                        out_hbm.at[c, pl.ds(s * 624, 624)])

    @pl.when(s == NS - 1)
    def _():
        pltpu.sync_copy(acc_sh.at[pl.ds((NS - 1) * 624, N - (NS - 1) * 624)],
                        out_hbm.at[c, pl.ds((NS - 1) * 624, N - (NS - 1) * 624)])


@functools.partial(
    pl.kernel,
    mesh=_sc_mesh(),
    out_type=(jax.ShapeDtypeStruct((NC, N, H), _F32),
              jax.ShapeDtypeStruct((NP,), _F32),
              jax.ShapeDtypeStruct((NP,), _F32)),
    scratch_types=[
        pltpu.VMEM((SEGT, CH), jnp.int32),
        pltpu.VMEM((SEGT, CH), jnp.int32),
        [pltpu.VMEM((CH, H), _F32)] * NB,
        [pltpu.VMEM((CH,), _F32)] * NB,
        pltpu.VMEM_SHARED((N + NPAD, H), _F32),
        pltpu.VMEM_SHARED((NP,), _F32),
        [pltpu.SemaphoreType.DMA] * NB,
        [pltpu.SemaphoreType.DMA] * NB,
        [pltpu.SemaphoreType.DMA] * NB,
        [pltpu.SemaphoreType.DMA] * NB,
    ],
)
def _att_sc(q_hbm, sm_hbm, src_hbm, dst_hbm, zeros_hbm, zrow_hbm,
            hraw_hbm, den0_hbm, den1_hbm, sidx_seg, didx_seg, rows, vals,
            acc_sh, den_sh, gsem, ssem, vgsem, vssem):
    c = lax.axis_index("c")
    s = lax.axis_index("s")
    w = c * NS + s
    kpw = NCHP // NW

    @pl.when(s == 0)
    def _():
        pltpu.sync_copy(zeros_hbm, acc_sh)
        pltpu.sync_copy(zrow_hbm, den_sh)

    plsc.subcore_barrier()

    def gat(m, i):
        pltpu.async_copy(q_hbm.at[sidx_seg.at[m]], rows[i], gsem[i])
        pltpu.async_copy(sm_hbm.at[sidx_seg.at[m]], vals[i], vgsem[i])

    def scat(m, i):
        pltpu.async_copy(rows[i], acc_sh.at[didx_seg.at[m]], ssem[i],
                         add=True)
        pltpu.async_copy(vals[i], den_sh.at[didx_seg.at[m]], vssem[i],
                         add=True)

    def wait_g(m, i):
        pltpu.make_async_copy(q_hbm.at[sidx_seg.at[m]], rows[i],
                              gsem[i]).wait()
        pltpu.make_async_copy(sm_hbm.at[sidx_seg.at[m]], vals[i],
                              vgsem[i]).wait()

    def wait_s(m, i):
        pltpu.make_async_copy(rows[i], acc_sh.at[didx_seg.at[m]],
                              ssem[i]).wait()
        pltpu.make_async_copy(vals[i], den_sh.at[didx_seg.at[m]],
                              vssem[i]).wait()

    def seg_body(g, carry):
        segbase = w * kpw + g * SEGT
        pltpu.sync_copy(src_hbm.at[pl.ds(segbase, SEGT)], sidx_seg)
        pltpu.sync_copy(dst_hbm.at[pl.ds(segbase, SEGT)], didx_seg)
        for i in range(NB):
            gat(i, i)

        def pair(m2, carry2):
            for i in range(NB):
                m = NB * m2 + i
                wait_g(m, i)
                scat(m, i)
            for i in range(NB):
                m = NB * m2 + i
                wait_s(m, i)
                gat(m + NB, i)
            return carry2

        lax.fori_loop(0, SEGT // NB - 1, pair, 0)
        for i in range(NB):
            m = SEGT - NB + i
            wait_g(m, i)
            scat(m, i)
        for i in range(NB):
            m = SEGT - NB + i
            wait_s(m, i)
        return carry

    lax.fori_loop(0, kpw // SEGT, seg_body, 0)
    plsc.subcore_barrier()
    _writeback_rows(acc_sh, hraw_hbm, c, s)
    _writeback_1d(den_sh, den0_hbm, den1_hbm, c, s)


def _lrelu(v):
    return jnp.where(v > 0, v, 0.2 * v)


def _full(shape):
    return pl.BlockSpec(shape, lambda i: tuple(0 for _ in shape))


def _xw_body(x_ref, w_ref, dg_ref, o_ref, xd_ref):
    o_ref[...] = jnp.dot(x_ref[...], w_ref[...],
                         preferred_element_type=_F32)
    deg = dg_ref[:, 0:1] + dg_ref[:, 1:2] + 1.0
    dinv = lax.rsqrt(deg)
    xd_ref[...] = x_ref[...] * dinv


def _xw_tc(x, Wcat, degT):
    return pl.pallas_call(
        _xw_body,
        grid=(GRID,),
        in_specs=[pl.BlockSpec((BN, D), lambda i: (i, 0)), _full((D, 2 * H)),
                  pl.BlockSpec((BN, 2), lambda i: (i, 0))],
        out_specs=[pl.BlockSpec((BN, 2 * H), lambda i: (i, 0)),
                   pl.BlockSpec((BN, D), lambda i: (i, 0))],
        out_shape=[jax.ShapeDtypeStruct((N, 2 * H), _F32),
                   jax.ShapeDtypeStruct((N, D), _F32)],
    )(x, Wcat, degT)


def _gcnep_body(up_ref, xw_ref, dg_ref, t_ref, wgo_ref, wgt_ref, bgo_ref,
                bgt_ref, wa_ref, ba_ref, d1w_ref, d1b_ref, d2w_ref, d2b_ref,
                d3w_ref, d3b_ref, ro_ref, rt_ref, pt_ref, sm_ref, q_ref):
    deg = dg_ref[:, 0:1] + dg_ref[:, 1:2] + 1.0
    dinv = lax.rsqrt(deg)
    dd = dinv * dinv
    xwo = xw_ref[:, :H]
    xwt = xw_ref[:, H:]
    u = up_ref[0] + up_ref[1]
    agg_o = jnp.dot(u, wgo_ref[...], preferred_element_type=_F32)
    agg_t = jnp.dot(u, wgt_ref[...], preferred_element_type=_F32)
    r_o = jnp.maximum(agg_o * dinv + xwo * dd + bgo_ref[...], 0.0)
    r_t = jnp.maximum(agg_t * dinv + xwt * dd + bgt_ref[...], 0.0)
    ro_ref[...] = r_o
    rt_ref[...] = r_t
    cat = jnp.concatenate([r_o, r_t], axis=1)
    sv = jnp.dot(cat, wa_ref[...], preferred_element_type=_F32) + ba_ref[...]
    s = _lrelu(sv)
    sm = jnp.where(t_ref[...] > 0, jnp.exp(jnp.minimum(s, 80.0)), 0.0)
    sm_ref[...] = sm
    q_ref[...] = sm * r_t
    v = _lrelu(jnp.dot(r_t, d1w_ref[...], preferred_element_type=_F32)
               + d1b_ref[...])
    v = _lrelu(jnp.dot(v, d2w_ref[...], preferred_element_type=_F32)
               + d2b_ref[...])
    pt_ref[...] = jax.nn.sigmoid(
        jnp.dot(v, d3w_ref[...], preferred_element_type=_F32) + d3b_ref[...])


def _gcnep_tc(Up, XW, degT, t2, Wgo, Wgt, bgo, bgt, Wa, ba, d1W, d1b,
              d2W, d2b, d3W, d3b):
    return pl.pallas_call(
        _gcnep_body,
        grid=(GRID,),
        in_specs=[
            pl.BlockSpec((NC, BN, H), lambda i: (0, i, 0)),
            pl.BlockSpec((BN, 2 * H), lambda i: (i, 0)),
            pl.BlockSpec((BN, 2), lambda i: (i, 0)),
            pl.BlockSpec((BN, 1), lambda i: (i, 0)),
            _full((H, H)), _full((H, H)),
            _full((1, H)), _full((1, H)), _full((2 * H, 1)), _full((1, 1)),
            _full((H, H)), _full((1, H)), _full((H, H)), _full((1, H)),
            _full((H, 1)), _full((1, 1)),
        ],
        out_specs=[
            pl.BlockSpec((BN, H), lambda i: (i, 0)),
            pl.BlockSpec((BN, H), lambda i: (i, 0)),
            pl.BlockSpec((BN, 1), lambda i: (i, 0)),
            pl.BlockSpec((BN, 1), lambda i: (i, 0)),
            pl.BlockSpec((BN, H), lambda i: (i, 0)),
        ],
        out_shape=[
            jax.ShapeDtypeStruct((N, H), _F32),
            jax.ShapeDtypeStruct((N, H), _F32),
            jax.ShapeDtypeStruct((N, 1), _F32),
            jax.ShapeDtypeStruct((N, 1), _F32),
            jax.ShapeDtypeStruct((N, H), _F32),
        ],
    )(Up, XW, degT, t2, Wgo, Wgt, bgo, bgt, Wa, ba, d1W, d1b, d2W, d2b,
      d3W, d3b)


def _mlp3_p(v, w1, b1, w2, b2, w3, b3):
    v = _lrelu(jnp.dot(v, w1[...], preferred_element_type=_F32) + b1[...])
    v = _lrelu(jnp.dot(v, w2[...], preferred_element_type=_F32) + b2[...])
    return jnp.dot(v, w3[...], preferred_element_type=_F32) + b3[...]


def _final_body(hraw_ref, den_ref, ro_ref, t_ref, weo_ref, weh_ref, be_ref,
                p1aW, p1ab, p1bW, p1bb, p1cW, p1cb,
                p0aW, p0ab, p0bW, p0bb, p0cW, p0cb,
                z2_ref, pred_ref):
    den = den_ref[:, 0:1] + den_ref[:, 1:2] + 1e-9
    h = (hraw_ref[0] + hraw_ref[1]) / den
    z2 = (jnp.dot(ro_ref[...], weo_ref[...], preferred_element_type=_F32)
          + jnp.dot(h, weh_ref[...], preferred_element_type=_F32)
          + be_ref[...])
    z2_ref[...] = z2
    p1 = _mlp3_p(z2, p1aW, p1ab, p1bW, p1bb, p1cW, p1cb)
    p0 = _mlp3_p(z2, p0aW, p0ab, p0bW, p0bb, p0cW, p0cb)
    pred_ref[...] = jnp.where(t_ref[...] > 0, p1, p0)


def _final_tc(hrawp, denT, ro, t2, Weo, Weh, be,
              p1aW, p1ab, p1bW, p1bb, p1cW, p1cb,
              p0aW, p0ab, p0bW, p0bb, p0cW, p0cb):
    return pl.pallas_call(
        _final_body,
        grid=(GRID,),
        in_specs=[
            pl.BlockSpec((NC, BN, H), lambda i: (0, i, 0)),
            pl.BlockSpec((BN, 2), lambda i: (i, 0)),
            pl.BlockSpec((BN, H), lambda i: (i, 0)),
            pl.BlockSpec((BN, 1), lambda i: (i, 0)),
            _full((H, H)), _full((H, H)), _full((1, H)),
            _full((H, H)), _full((1, H)), _full((H, H)), _full((1, H)),
            _full((H, 1)), _full((1, 1)),
            _full((H, H)), _full((1, H)), _full((H, H)), _full((1, H)),
            _full((H, 1)), _full((1, 1)),
        ],
        out_specs=[pl.BlockSpec((BN, H), lambda i: (i, 0)),
                   pl.BlockSpec((BN, 1), lambda i: (i, 0))],
        out_shape=[jax.ShapeDtypeStruct((N, H), _F32),
                   jax.ShapeDtypeStruct((N, 1), _F32)],
    )(hrawp, denT, ro, t2, Weo, Weh, be,
      p1aW, p1ab, p1bW, p1bb, p1cW, p1cb,
      p0aW, p0ab, p0bW, p0bb, p0cW, p0cb)


@jax.jit
def kernel(x, t, z, edge_index, Wgo, bgo, Wgt, bgt, Wa, ba, We, be,
           d1W, d1b, d2W, d2b, d3W, d3b, p1aW, p1ab, p1bW, p1bb, p1cW, p1cb,
           p0aW, p0ab, p0bW, p0bb, p0cW, p0cb):
    npad = EP - E
    ar = lax.iota(jnp.int32, npad)
    src2 = jnp.concatenate([edge_index[0], ar % 128]).reshape(NCHP, CH)
    dst2 = jnp.concatenate([edge_index[1], N + (ar % NPAD)]).reshape(NCHP, CH)
    t2 = t[:, None]
    zeros2 = jnp.zeros((N + NPAD, H), _F32)
    zrow = jnp.zeros((NP,), _F32)
    Wcat = jnp.concatenate([Wgo, Wgt], axis=1)

    deg0, deg1 = _deg_sc(dst2, zrow)
    degT = jnp.stack([deg0[:N], deg1[:N]], axis=1)
    XW, xd = _xw_tc(x, Wcat, degT)
    Up = _agg_sc(xd, src2, dst2, zeros2)
    ro, rt, pt, sm2, q = _gcnep_tc(
        Up, XW, degT, t2, Wgo, Wgt, bgo[None, :], bgt[None, :], Wa,
        ba[None, :], d1W, d1b[None, :], d2W, d2b[None, :], d3W, d3b[None, :])
    hrawp, den0, den1 = _att_sc(q, sm2[:, 0], src2, dst2, zeros2, zrow)
    denT = jnp.stack([den0[:N], den1[:N]], axis=1)
    z2, pred = _final_tc(
        hrawp, denT, ro, t2, We[:H], We[H:], be[None, :],
        p1aW, p1ab[None, :], p1bW, p1bb[None, :], p1cW, p1cb[None, :],
        p0aW, p0ab[None, :], p0bW, p0bb[None, :], p0cW, p0cb[None, :])
    return (pt, pred, z2)


# speedup vs baseline: 1.0202x; 1.0202x over previous
import functools

import jax
import jax.numpy as jnp
from jax import lax
from jax.experimental import pallas as pl
from jax.experimental.pallas import tpu as pltpu
from jax.experimental.pallas import tpu_sc as plsc

N = 10000
E = 320000
D = 128
H = 128
CH = 128
NCH = E // CH
NS = 16
NC = 2
NW = NC * NS
BN = 1000
GRID = N // BN
NP = 10240
NCHP = 2560
EP = NCHP * CH
NPAD = 16
NB = 2
SEGA = 32
SEGT = 40

_F32 = jnp.float32


def _sc_mesh():
    return plsc.VectorSubcoreMesh(core_axis_name="c", subcore_axis_name="s")


@functools.partial(
    pl.kernel,
    mesh=_sc_mesh(),
    out_type=(jax.ShapeDtypeStruct((NP,), _F32),
              jax.ShapeDtypeStruct((NP,), _F32)),
    scratch_types=[
        pltpu.VMEM((NCHP // NW, CH), jnp.int32),
        pltpu.VMEM((CH,), _F32),
        pltpu.VMEM_SHARED((NP,), _F32),
        pltpu.SemaphoreType.DMA,
        pltpu.SemaphoreType.DMA,
        pltpu.SemaphoreType.DMA,
        pltpu.SemaphoreType.DMA,
    ],
)
def _deg_sc(dst_hbm, zrow_hbm, out0_hbm, out1_hbm, didx_all, ones_v, acc_sh,
            *sems):
    c = lax.axis_index("c")
    s = lax.axis_index("s")
    w = c * NS + s
    kpw = NCHP // NW
    one = jnp.full((16,), 1.0, _F32)
    for i in range(CH // 16):
        ones_v[pl.ds(i * 16, 16)] = one
    pltpu.sync_copy(dst_hbm.at[pl.ds(w * kpw, kpw)], didx_all)

    @pl.when(s == 0)
    def _():
        pltpu.sync_copy(zrow_hbm, acc_sh)

    plsc.subcore_barrier()

    def scat(j, i):
        return pltpu.async_copy(ones_v, acc_sh.at[didx_all.at[j]], sems[i],
                                add=True)

    for i in range(NB):
        scat(i, i)

    def body(k4, carry):
        for i in range(NB):
            j = NB * k4 + i
            pltpu.make_async_copy(ones_v, acc_sh.at[didx_all.at[j]],
                                  sems[i]).wait()
            scat(j + NB, i)
        return carry

    lax.fori_loop(0, kpw // NB - 1, body, 0)
    for i in range(NB):
        j = kpw - NB + i
        pltpu.make_async_copy(ones_v, acc_sh.at[didx_all.at[j]],
                              sems[i]).wait()
    plsc.subcore_barrier()
    _writeback_1d(acc_sh, out0_hbm, out1_hbm, c, s)


def _writeback_1d(acc_sh, out0_hbm, out1_hbm, c, s):
    def wb(out_hbm):
        pltpu.sync_copy(acc_sh.at[pl.ds(s * 640, 640)],
                        out_hbm.at[pl.ds(s * 640, 640)])

    @pl.when(c == 0)
    def _():
        wb(out0_hbm)

    @pl.when(c == 1)
    def _():
        wb(out1_hbm)


@functools.partial(
    pl.kernel,
    mesh=_sc_mesh(),
    out_type=jax.ShapeDtypeStruct((NC, N, H), _F32),
    scratch_types=[
        pltpu.VMEM((SEGT, CH), jnp.int32),
        pltpu.VMEM((SEGT, CH), jnp.int32),
        [pltpu.VMEM((CH, H), _F32)] * NB,
        pltpu.VMEM_SHARED((N + NPAD, H), _F32),
        [pltpu.SemaphoreType.DMA] * NB,
        [pltpu.SemaphoreType.DMA] * NB,
    ],
)
def _agg_sc(xd_hbm, src_hbm, dst_hbm, zeros_hbm, out_hbm,
            sidx_seg, didx_seg, rows, acc_sh, gsem, ssem):
    c = lax.axis_index("c")
    s = lax.axis_index("s")
    w = c * NS + s
    kpw = NCHP // NW

    @pl.when(s == 0)
    def _():
        pltpu.sync_copy(zeros_hbm, acc_sh)

    plsc.subcore_barrier()
    _seg_pipeline(xd_hbm, acc_sh, src_hbm, dst_hbm, sidx_seg, didx_seg,
                  rows, gsem, ssem, w * kpw, kpw // SEGT, SEGT)
    plsc.subcore_barrier()
    _writeback_rows(acc_sh, out_hbm, c, s)


def _seg_pipeline(table, acc_sh, src_hbm, dst_hbm, sidx_seg, didx_seg,
                  rows, gsem, ssem, tile_base, nseg, seg):
    def gat(m, i):
        pltpu.async_copy(table.at[sidx_seg.at[m]], rows[i], gsem[i])

    def wait_g(m, i):
        pltpu.make_async_copy(table.at[sidx_seg.at[m]], rows[i],
                              gsem[i]).wait()

    def scat(m, i):
        pltpu.async_copy(rows[i], acc_sh.at[didx_seg.at[m]], ssem[i],
                         add=True)

    def wait_s(m, i):
        pltpu.make_async_copy(rows[i], acc_sh.at[didx_seg.at[m]],
                              ssem[i]).wait()

    def seg_body(g, carry):
        segbase = tile_base + g * seg
        pltpu.sync_copy(src_hbm.at[pl.ds(segbase, seg)], sidx_seg)
        pltpu.sync_copy(dst_hbm.at[pl.ds(segbase, seg)], didx_seg)
        for i in range(NB):
            gat(i, i)

        def pair(m2, carry2):
            for i in range(NB):
                m = NB * m2 + i
                wait_g(m, i)
                scat(m, i)
            for i in range(NB):
                m = NB * m2 + i
                wait_s(m, i)
                gat(m + NB, i)
            return carry2

        lax.fori_loop(0, seg // NB - 1, pair, 0)
        for i in range(NB):
            m = seg - NB + i
            wait_g(m, i)
            scat(m, i)
        for i in range(NB):
            m = seg - NB + i
            wait_s(m, i)
        return carry

    lax.fori_loop(0, nseg, seg_body, 0)


def _writeback_rows(acc_sh, out_hbm, c, s):
    @pl.when(s < NS - 1)
    def _():
        pltpu.sync_copy(acc_sh.at[pl.ds(s * 624, 624)],
                        out_hbm.at[c, pl.ds(s * 624, 624)])

    @pl.when(s == NS - 1)
    def _():
        pltpu.sync_copy(acc_sh.at[pl.ds((NS - 1) * 624, N - (NS - 1) * 624)],
                        out_hbm.at[c, pl.ds((NS - 1) * 624, N - (NS - 1) * 624)])


@functools.partial(
    pl.kernel,
    mesh=_sc_mesh(),
    out_type=(jax.ShapeDtypeStruct((NC, N, H), _F32),
              jax.ShapeDtypeStruct((NP,), _F32),
              jax.ShapeDtypeStruct((NP,), _F32)),
    scratch_types=[
        pltpu.VMEM((SEGT, CH), jnp.int32),
        pltpu.VMEM((SEGT, CH), jnp.int32),
        [pltpu.VMEM((CH, H), _F32)] * NB,
        [pltpu.VMEM((CH,), _F32)] * NB,
        pltpu.VMEM_SHARED((N + NPAD, H), _F32),
        pltpu.VMEM_SHARED((NP,), _F32),
        [pltpu.SemaphoreType.DMA] * NB,
        [pltpu.SemaphoreType.DMA] * NB,
        [pltpu.SemaphoreType.DMA] * NB,
        [pltpu.SemaphoreType.DMA] * NB,
    ],
)
def _att_sc(q_hbm, sm_hbm, src_hbm, dst_hbm, zeros_hbm, zrow_hbm,
            hraw_hbm, den0_hbm, den1_hbm, sidx_seg, didx_seg, rows, vals,
            acc_sh, den_sh, gsem, ssem, vgsem, vssem):
    c = lax.axis_index("c")
    s = lax.axis_index("s")
    w = c * NS + s
    kpw = NCHP // NW

    @pl.when(s == 0)
    def _():
        pltpu.sync_copy(zeros_hbm, acc_sh)
        pltpu.sync_copy(zrow_hbm, den_sh)

    plsc.subcore_barrier()

    def gat(m, i):
        pltpu.async_copy(q_hbm.at[sidx_seg.at[m]], rows[i], gsem[i])
        pltpu.async_copy(sm_hbm.at[sidx_seg.at[m]], vals[i], vgsem[i])

    def scat(m, i):
        pltpu.async_copy(rows[i], acc_sh.at[didx_seg.at[m]], ssem[i],
                         add=True)
        pltpu.async_copy(vals[i], den_sh.at[didx_seg.at[m]], vssem[i],
                         add=True)

    def wait_g(m, i):
        pltpu.make_async_copy(q_hbm.at[sidx_seg.at[m]], rows[i],
                              gsem[i]).wait()
        pltpu.make_async_copy(sm_hbm.at[sidx_seg.at[m]], vals[i],
                              vgsem[i]).wait()

    def wait_s(m, i):
        pltpu.make_async_copy(rows[i], acc_sh.at[didx_seg.at[m]],
                              ssem[i]).wait()
        pltpu.make_async_copy(vals[i], den_sh.at[didx_seg.at[m]],
                              vssem[i]).wait()

    def seg_body(g, carry):
        segbase = w * kpw + g * SEGT
        pltpu.sync_copy(src_hbm.at[pl.ds(segbase, SEGT)], sidx_seg)
        pltpu.sync_copy(dst_hbm.at[pl.ds(segbase, SEGT)], didx_seg)
        for i in range(NB):
            gat(i, i)

        def pair(m2, carry2):
            for i in range(NB):
                m = NB * m2 + i
                wait_g(m, i)
                scat(m, i)
            for i in range(NB):
                m = NB * m2 + i
                wait_s(m, i)
                gat(m + NB, i)
            return carry2

        lax.fori_loop(0, SEGT // NB - 1, pair, 0)
        for i in range(NB):
            m = SEGT - NB + i
            wait_g(m, i)
            scat(m, i)
        for i in range(NB):
            m = SEGT - NB + i
            wait_s(m, i)
        return carry

    lax.fori_loop(0, kpw // SEGT, seg_body, 0)
    plsc.subcore_barrier()
    _writeback_rows(acc_sh, hraw_hbm, c, s)
    _writeback_1d(den_sh, den0_hbm, den1_hbm, c, s)


def _lrelu(v):
    return jnp.where(v > 0, v, 0.2 * v)


def _full(shape):
    return pl.BlockSpec(shape, lambda i: tuple(0 for _ in shape))


def _xw_body(x_ref, w_ref, dg_ref, o_ref, xd_ref):
    o_ref[...] = jnp.dot(x_ref[...], w_ref[...],
                         preferred_element_type=_F32)
    deg = dg_ref[:, 0:1] + dg_ref[:, 1:2] + 1.0
    dinv = lax.rsqrt(deg)
    xd_ref[...] = x_ref[...] * dinv


def _xw_tc(x, Wcat, degT):
    return pl.pallas_call(
        _xw_body,
        grid=(GRID,),
        in_specs=[pl.BlockSpec((BN, D), lambda i: (i, 0)), _full((D, 2 * H)),
                  pl.BlockSpec((BN, 2), lambda i: (i, 0))],
        out_specs=[pl.BlockSpec((BN, 2 * H), lambda i: (i, 0)),
                   pl.BlockSpec((BN, D), lambda i: (i, 0))],
        out_shape=[jax.ShapeDtypeStruct((N, 2 * H), _F32),
                   jax.ShapeDtypeStruct((N, D), _F32)],
    )(x, Wcat, degT)


def _gcnep_body(up_ref, xw_ref, dg_ref, t_ref, wgo_ref, wgt_ref, bgo_ref,
                bgt_ref, wa_ref, ba_ref, d1w_ref, d1b_ref, d2w_ref, d2b_ref,
                d3w_ref, d3b_ref, ro_ref, rt_ref, pt_ref, sm_ref, q_ref):
    deg = dg_ref[:, 0:1] + dg_ref[:, 1:2] + 1.0
    dinv = lax.rsqrt(deg)
    dd = dinv * dinv
    xwo = xw_ref[:, :H]
    xwt = xw_ref[:, H:]
    u = up_ref[0] + up_ref[1]
    agg_o = jnp.dot(u, wgo_ref[...], preferred_element_type=_F32)
    agg_t = jnp.dot(u, wgt_ref[...], preferred_element_type=_F32)
    r_o = jnp.maximum(agg_o * dinv + xwo * dd + bgo_ref[...], 0.0)
    r_t = jnp.maximum(agg_t * dinv + xwt * dd + bgt_ref[...], 0.0)
    ro_ref[...] = r_o
    rt_ref[...] = r_t
    cat = jnp.concatenate([r_o, r_t], axis=1)
    sv = jnp.dot(cat, wa_ref[...], preferred_element_type=_F32) + ba_ref[...]
    s = _lrelu(sv)
    sm = jnp.where(t_ref[...] > 0, jnp.exp(jnp.minimum(s, 80.0)), 0.0)
    sm_ref[...] = sm
    q_ref[...] = sm * r_t
    v = _lrelu(jnp.dot(r_t, d1w_ref[...], preferred_element_type=_F32)
               + d1b_ref[...])
    v = _lrelu(jnp.dot(v, d2w_ref[...], preferred_element_type=_F32)
               + d2b_ref[...])
    pt_ref[...] = jax.nn.sigmoid(
        jnp.dot(v, d3w_ref[...], preferred_element_type=_F32) + d3b_ref[...])


def _gcnep_tc(Up, XW, degT, t2, Wgo, Wgt, bgo, bgt, Wa, ba, d1W, d1b,
              d2W, d2b, d3W, d3b):
    return pl.pallas_call(
        _gcnep_body,
        grid=(GRID,),
        in_specs=[
            pl.BlockSpec((NC, BN, H), lambda i: (0, i, 0)),
            pl.BlockSpec((BN, 2 * H), lambda i: (i, 0)),
            pl.BlockSpec((BN, 2), lambda i: (i, 0)),
            pl.BlockSpec((BN, 1), lambda i: (i, 0)),
            _full((H, H)), _full((H, H)),
            _full((1, H)), _full((1, H)), _full((2 * H, 1)), _full((1, 1)),
            _full((H, H)), _full((1, H)), _full((H, H)), _full((1, H)),
            _full((H, 1)), _full((1, 1)),
        ],
        out_specs=[
            pl.BlockSpec((BN, H), lambda i: (i, 0)),
            pl.BlockSpec((BN, H), lambda i: (i, 0)),
            pl.BlockSpec((BN, 1), lambda i: (i, 0)),
            pl.BlockSpec((BN, 1), lambda i: (i, 0)),
            pl.BlockSpec((BN, H), lambda i: (i, 0)),
        ],
        out_shape=[
            jax.ShapeDtypeStruct((N, H), _F32),
            jax.ShapeDtypeStruct((N, H), _F32),
            jax.ShapeDtypeStruct((N, 1), _F32),
            jax.ShapeDtypeStruct((N, 1), _F32),
            jax.ShapeDtypeStruct((N, H), _F32),
        ],
    )(Up, XW, degT, t2, Wgo, Wgt, bgo, bgt, Wa, ba, d1W, d1b, d2W, d2b,
      d3W, d3b)


def _mlp3_p(v, w1, b1, w2, b2, w3, b3):
    v = _lrelu(jnp.dot(v, w1[...], preferred_element_type=_F32) + b1[...])
    v = _lrelu(jnp.dot(v, w2[...], preferred_element_type=_F32) + b2[...])
    return jnp.dot(v, w3[...], preferred_element_type=_F32) + b3[...]


def _final_body(hraw_ref, den_ref, ro_ref, t_ref, weo_ref, weh_ref, be_ref,
                p1aW, p1ab, p1bW, p1bb, p1cW, p1cb,
                p0aW, p0ab, p0bW, p0bb, p0cW, p0cb,
                z2_ref, pred_ref):
    den = den_ref[:, 0:1] + den_ref[:, 1:2] + 1e-9
    h = (hraw_ref[0] + hraw_ref[1]) / den
    z2 = (jnp.dot(ro_ref[...], weo_ref[...], preferred_element_type=_F32)
          + jnp.dot(h, weh_ref[...], preferred_element_type=_F32)
          + be_ref[...])
    z2_ref[...] = z2
    p1 = _mlp3_p(z2, p1aW, p1ab, p1bW, p1bb, p1cW, p1cb)
    p0 = _mlp3_p(z2, p0aW, p0ab, p0bW, p0bb, p0cW, p0cb)
    pred_ref[...] = jnp.where(t_ref[...] > 0, p1, p0)


def _final_tc(hrawp, denT, ro, t2, Weo, Weh, be,
              p1aW, p1ab, p1bW, p1bb, p1cW, p1cb,
              p0aW, p0ab, p0bW, p0bb, p0cW, p0cb):
    return pl.pallas_call(
        _final_body,
        grid=(GRID,),
        in_specs=[
            pl.BlockSpec((NC, BN, H), lambda i: (0, i, 0)),
            pl.BlockSpec((BN, 2), lambda i: (i, 0)),
            pl.BlockSpec((BN, H), lambda i: (i, 0)),
            pl.BlockSpec((BN, 1), lambda i: (i, 0)),
            _full((H, H)), _full((H, H)), _full((1, H)),
            _full((H, H)), _full((1, H)), _full((H, H)), _full((1, H)),
            _full((H, 1)), _full((1, 1)),
            _full((H, H)), _full((1, H)), _full((H, H)), _full((1, H)),
            _full((H, 1)), _full((1, 1)),
        ],
        out_specs=[pl.BlockSpec((BN, H), lambda i: (i, 0)),
                   pl.BlockSpec((BN, 1), lambda i: (i, 0))],
        out_shape=[jax.ShapeDtypeStruct((N, H), _F32),
                   jax.ShapeDtypeStruct((N, 1), _F32)],
    )(hrawp, denT, ro, t2, Weo, Weh, be,
      p1aW, p1ab, p1bW, p1bb, p1cW, p1cb,
      p0aW, p0ab, p0bW, p0bb, p0cW, p0cb)


@jax.jit
def kernel(x, t, z, edge_index, Wgo, bgo, Wgt, bgt, Wa, ba, We, be,
           d1W, d1b, d2W, d2b, d3W, d3b, p1aW, p1ab, p1bW, p1bb, p1cW, p1cb,
           p0aW, p0ab, p0bW, p0bb, p0cW, p0cb):
    npad = EP - E
    ar = lax.iota(jnp.int32, npad)
    src2 = jnp.concatenate([edge_index[0], ar % 128]).reshape(NCHP, CH)
    dst2 = jnp.concatenate([edge_index[1], N + (ar % NPAD)]).reshape(NCHP, CH)
    t2 = t[:, None]
    zeros2 = jnp.zeros((N + NPAD, H), _F32)
    zrow = jnp.zeros((NP,), _F32)
    Wcat = jnp.concatenate([Wgo, Wgt], axis=1)

    deg0, deg1 = _deg_sc(dst2, zrow)
    degT = jnp.stack([deg0[:N], deg1[:N]], axis=1)
    XW, xd = _xw_tc(x, Wcat, degT)
    Up = _agg_sc(xd, src2, dst2, zeros2)
    ro, rt, pt, sm2, q = _gcnep_tc(
        Up, XW, degT, t2, Wgo, Wgt, bgo[None, :], bgt[None, :], Wa,
        ba[None, :], d1W, d1b[None, :], d2W, d2b[None, :], d3W, d3b[None, :])
    hrawp, den0, den1 = _att_sc(q, sm2[:, 0], src2, dst2, zeros2, zrow)
    denT = jnp.stack([den0[:N], den1[:N]], axis=1)
    z2, pred = _final_tc(
        hrawp, denT, ro, t2, We[:H], We[H:], be[None, :],
        p1aW, p1ab[None, :], p1bW, p1bb[None, :], p1cW, p1cb[None, :],
        p0aW, p0ab[None, :], p0bW, p0bb[None, :], p0cW, p0cb[None, :])
    return (pt, pred, z2)


# speedup vs baseline: 1.0205x; 1.0003x over previous
import functools

import jax
import jax.numpy as jnp
from jax import lax
from jax.experimental import pallas as pl
from jax.experimental.pallas import tpu as pltpu
from jax.experimental.pallas import tpu_sc as plsc

N = 10000
E = 320000
D = 128
H = 128
CH = 128
NS = 16
NC = 2
NW = NC * NS
BN = 1000
GRID = N // BN
NP = 10240
NCHP = 2560
EP = NCHP * CH
NPAD = 16
NB = 2
SEGT = 40

_F32 = jnp.float32


def _sc_mesh():
    return plsc.VectorSubcoreMesh(core_axis_name="c", subcore_axis_name="s")


@functools.partial(
    pl.kernel,
    mesh=_sc_mesh(),
    out_type=(jax.ShapeDtypeStruct((NP,), _F32),
              jax.ShapeDtypeStruct((NP,), _F32)),
    scratch_types=[
        pltpu.VMEM((NCHP // NW, CH), jnp.int32),
        pltpu.VMEM((CH,), _F32),
        pltpu.VMEM_SHARED((NP,), _F32),
        pltpu.SemaphoreType.DMA,
        pltpu.SemaphoreType.DMA,
        pltpu.SemaphoreType.DMA,
        pltpu.SemaphoreType.DMA,
    ],
)
def _deg_sc(dst_hbm, zrow_hbm, out0_hbm, out1_hbm, didx_all, ones_v, acc_sh,
            *sems):
    c = lax.axis_index("c")
    s = lax.axis_index("s")
    w = c * NS + s
    kpw = NCHP // NW
    one = jnp.full((16,), 1.0, _F32)
    for i in range(CH // 16):
        ones_v[pl.ds(i * 16, 16)] = one
    pltpu.sync_copy(dst_hbm.at[pl.ds(w * kpw, kpw)], didx_all)

    @pl.when(s == 0)
    def _():
        pltpu.sync_copy(zrow_hbm, acc_sh)

    plsc.subcore_barrier()

    def scat(j, i):
        return pltpu.async_copy(ones_v, acc_sh.at[didx_all.at[j]], sems[i],
                                add=True)

    for i in range(NB):
        scat(i, i)

    def body(k4, carry):
        for i in range(NB):
            j = NB * k4 + i
            pltpu.make_async_copy(ones_v, acc_sh.at[didx_all.at[j]],
                                  sems[i]).wait()
            scat(j + NB, i)
        return carry

    lax.fori_loop(0, kpw // NB - 1, body, 0)
    for i in range(NB):
        j = kpw - NB + i
        pltpu.make_async_copy(ones_v, acc_sh.at[didx_all.at[j]],
                              sems[i]).wait()
    plsc.subcore_barrier()
    _writeback_1d(acc_sh, out0_hbm, out1_hbm, c, s)


def _writeback_1d(acc_sh, out0_hbm, out1_hbm, c, s):
    def wb(out_hbm):
        pltpu.sync_copy(acc_sh.at[pl.ds(s * 640, 640)],
                        out_hbm.at[pl.ds(s * 640, 640)])

    @pl.when(c == 0)
    def _():
        wb(out0_hbm)

    @pl.when(c == 1)
    def _():
        wb(out1_hbm)


@functools.partial(
    pl.kernel,
    mesh=_sc_mesh(),
    out_type=jax.ShapeDtypeStruct((NC, N, H), _F32),
    scratch_types=[
        pltpu.VMEM((SEGT, CH), jnp.int32),
        pltpu.VMEM((SEGT, CH), jnp.int32),
        [pltpu.VMEM((CH, H), _F32)] * NB,
        pltpu.VMEM_SHARED((N + NPAD, H), _F32),
        [pltpu.SemaphoreType.DMA] * NB,
        [pltpu.SemaphoreType.DMA] * NB,
    ],
)
def _agg_sc(xd_hbm, src_hbm, dst_hbm, zeros_hbm, out_hbm,
            sidx_seg, didx_seg, rows, acc_sh, gsem, ssem):
    c = lax.axis_index("c")
    s = lax.axis_index("s")
    w = c * NS + s
    kpw = NCHP // NW

    @pl.when(s == 0)
    def _():
        pltpu.sync_copy(zeros_hbm, acc_sh)

    plsc.subcore_barrier()
    _seg_pipeline(xd_hbm, acc_sh, src_hbm, dst_hbm, sidx_seg, didx_seg,
                  rows, gsem, ssem, w * kpw, kpw // SEGT, SEGT)
    plsc.subcore_barrier()
    _writeback_rows(acc_sh, out_hbm, c, s)


def _seg_pipeline(table, acc_sh, src_hbm, dst_hbm, sidx_seg, didx_seg,
                  rows, gsem, ssem, tile_base, nseg, seg):
    def gat(m, i):
        pltpu.async_copy(table.at[sidx_seg.at[m]], rows[i], gsem[i])

    def wait_g(m, i):
        pltpu.make_async_copy(table.at[sidx_seg.at[m]], rows[i],
                              gsem[i]).wait()

    def scat(m, i):
        pltpu.async_copy(rows[i], acc_sh.at[didx_seg.at[m]], ssem[i],
                         add=True)

    def wait_s(m, i):
        pltpu.make_async_copy(rows[i], acc_sh.at[didx_seg.at[m]],
                              ssem[i]).wait()

    def seg_body(g, carry):
        segbase = tile_base + g * seg
        pltpu.sync_copy(src_hbm.at[pl.ds(segbase, seg)], sidx_seg)
        pltpu.sync_copy(dst_hbm.at[pl.ds(segbase, seg)], didx_seg)
        for i in range(NB):
            gat(i, i)

        def pair(m2, carry2):
            for i in range(NB):
                m = NB * m2 + i
                wait_g(m, i)
                scat(m, i)
            for i in range(NB):
                m = NB * m2 + i
                wait_s(m, i)
                gat(m + NB, i)
            return carry2

        lax.fori_loop(0, seg // NB - 1, pair, 0)
        for i in range(NB):
            m = seg - NB + i
            wait_g(m, i)
            scat(m, i)
        for i in range(NB):
            m = seg - NB + i
            wait_s(m, i)
        return carry

    lax.fori_loop(0, nseg, seg_body, 0)


def _writeback_rows(acc_sh, out_hbm, c, s):
    @pl.when(s < NS - 1)
    def _():
        pltpu.sync_copy(acc_sh.at[pl.ds(s * 624, 624)],
                        out_hbm.at[c, pl.ds(s * 624, 624)])

    @pl.when(s == NS - 1)
    def _():
        pltpu.sync_copy(acc_sh.at[pl.ds((NS - 1) * 624, N - (NS - 1) * 624)],
                        out_hbm.at[c, pl.ds((NS - 1) * 624, N - (NS - 1) * 624)])


@functools.partial(
    pl.kernel,
    mesh=_sc_mesh(),
    out_type=(jax.ShapeDtypeStruct((NC, N, H), _F32),
              jax.ShapeDtypeStruct((NP,), _F32),
              jax.ShapeDtypeStruct((NP,), _F32)),
    scratch_types=[
        pltpu.VMEM((SEGT, CH), jnp.int32),
        pltpu.VMEM((SEGT, CH), jnp.int32),
        [pltpu.VMEM((CH, H), _F32)] * NB,
        [pltpu.VMEM((CH,), _F32)] * NB,
        pltpu.VMEM_SHARED((N + NPAD, H), _F32),
        pltpu.VMEM_SHARED((NP,), _F32),
        [pltpu.SemaphoreType.DMA] * NB,
        [pltpu.SemaphoreType.DMA] * NB,
        [pltpu.SemaphoreType.DMA] * NB,
        [pltpu.SemaphoreType.DMA] * NB,
    ],
)
def _att_sc(q_hbm, sm_hbm, src_hbm, dst_hbm, zeros_hbm, zrow_hbm,
            hraw_hbm, den0_hbm, den1_hbm, sidx_seg, didx_seg, rows, vals,
            acc_sh, den_sh, gsem, ssem, vgsem, vssem):
    c = lax.axis_index("c")
    s = lax.axis_index("s")
    w = c * NS + s
    kpw = NCHP // NW

    @pl.when(s == 0)
    def _():
        pltpu.sync_copy(zeros_hbm, acc_sh)
        pltpu.sync_copy(zrow_hbm, den_sh)

    plsc.subcore_barrier()

    def gat(m, i):
        pltpu.async_copy(q_hbm.at[sidx_seg.at[m]], rows[i], gsem[i])
        pltpu.async_copy(sm_hbm.at[sidx_seg.at[m]], vals[i], vgsem[i])

    def scat(m, i):
        pltpu.async_copy(rows[i], acc_sh.at[didx_seg.at[m]], ssem[i],
                         add=True)
        pltpu.async_copy(vals[i], den_sh.at[didx_seg.at[m]], vssem[i],
                         add=True)

    def wait_g(m, i):
        pltpu.make_async_copy(q_hbm.at[sidx_seg.at[m]], rows[i],
                              gsem[i]).wait()
        pltpu.make_async_copy(sm_hbm.at[sidx_seg.at[m]], vals[i],
                              vgsem[i]).wait()

    def wait_s(m, i):
        pltpu.make_async_copy(rows[i], acc_sh.at[didx_seg.at[m]],
                              ssem[i]).wait()
        pltpu.make_async_copy(vals[i], den_sh.at[didx_seg.at[m]],
                              vssem[i]).wait()

    def seg_body(g, carry):
        segbase = w * kpw + g * SEGT
        pltpu.sync_copy(src_hbm.at[pl.ds(segbase, SEGT)], sidx_seg)
        pltpu.sync_copy(dst_hbm.at[pl.ds(segbase, SEGT)], didx_seg)
        for i in range(NB):
            gat(i, i)

        def pair(m2, carry2):
            for i in range(NB):
                m = NB * m2 + i
                wait_g(m, i)
                scat(m, i)
            for i in range(NB):
                m = NB * m2 + i
                wait_s(m, i)
                gat(m + NB, i)
            return carry2

        lax.fori_loop(0, SEGT // NB - 1, pair, 0)
        for i in range(NB):
            m = SEGT - NB + i
            wait_g(m, i)
            scat(m, i)
        for i in range(NB):
            m = SEGT - NB + i
            wait_s(m, i)
        return carry

    lax.fori_loop(0, kpw // SEGT, seg_body, 0)
    plsc.subcore_barrier()
    _writeback_rows(acc_sh, hraw_hbm, c, s)
    _writeback_1d(den_sh, den0_hbm, den1_hbm, c, s)


def _lrelu(v):
    return jnp.where(v > 0, v, 0.2 * v)


def _full(shape):
    return pl.BlockSpec(shape, lambda i: tuple(0 for _ in shape))


def _xw_body(x_ref, w_ref, dg_ref, o_ref, xd_ref):
    o_ref[...] = jnp.dot(x_ref[...], w_ref[...],
                         preferred_element_type=_F32)
    deg = dg_ref[:, 0:1] + dg_ref[:, 1:2] + 1.0
    dinv = lax.rsqrt(deg)
    xd_ref[...] = x_ref[...] * dinv


def _xw_tc(x, Wcat, degT):
    return pl.pallas_call(
        _xw_body,
        grid=(GRID,),
        in_specs=[pl.BlockSpec((BN, D), lambda i: (i, 0)), _full((D, 2 * H)),
                  pl.BlockSpec((BN, 2), lambda i: (i, 0))],
        out_specs=[pl.BlockSpec((BN, 2 * H), lambda i: (i, 0)),
                   pl.BlockSpec((BN, D), lambda i: (i, 0))],
        out_shape=[jax.ShapeDtypeStruct((N, 2 * H), _F32),
                   jax.ShapeDtypeStruct((N, D), _F32)],
    )(x, Wcat, degT)


def _gcnep_body(up_ref, xw_ref, dg_ref, t_ref, wgo_ref, wgt_ref, bgo_ref,
                bgt_ref, wa_ref, ba_ref, d1w_ref, d1b_ref, d2w_ref, d2b_ref,
                d3w_ref, d3b_ref, ro_ref, rt_ref, pt_ref, sm_ref, q_ref):
    deg = dg_ref[:, 0:1] + dg_ref[:, 1:2] + 1.0
    dinv = lax.rsqrt(deg)
    dd = dinv * dinv
    xwo = xw_ref[:, :H]
    xwt = xw_ref[:, H:]
    u = up_ref[0] + up_ref[1]
    agg_o = jnp.dot(u, wgo_ref[...], preferred_element_type=_F32)
    agg_t = jnp.dot(u, wgt_ref[...], preferred_element_type=_F32)
    r_o = jnp.maximum(agg_o * dinv + xwo * dd + bgo_ref[...], 0.0)
    r_t = jnp.maximum(agg_t * dinv + xwt * dd + bgt_ref[...], 0.0)
    ro_ref[...] = r_o
    rt_ref[...] = r_t
    cat = jnp.concatenate([r_o, r_t], axis=1)
    sv = jnp.dot(cat, wa_ref[...], preferred_element_type=_F32) + ba_ref[...]
    s = _lrelu(sv)
    sm = jnp.where(t_ref[...] > 0, jnp.exp(jnp.minimum(s, 80.0)), 0.0)
    sm_ref[...] = sm
    q_ref[...] = sm * r_t
    v = _lrelu(jnp.dot(r_t, d1w_ref[...], preferred_element_type=_F32)
               + d1b_ref[...])
    v = _lrelu(jnp.dot(v, d2w_ref[...], preferred_element_type=_F32)
               + d2b_ref[...])
    pt_ref[...] = jax.nn.sigmoid(
        jnp.dot(v, d3w_ref[...], preferred_element_type=_F32) + d3b_ref[...])


def _gcnep_tc(Up, XW, degT, t2, Wgo, Wgt, bgo, bgt, Wa, ba, d1W, d1b,
              d2W, d2b, d3W, d3b):
    return pl.pallas_call(
        _gcnep_body,
        grid=(GRID,),
        in_specs=[
            pl.BlockSpec((NC, BN, H), lambda i: (0, i, 0)),
            pl.BlockSpec((BN, 2 * H), lambda i: (i, 0)),
            pl.BlockSpec((BN, 2), lambda i: (i, 0)),
            pl.BlockSpec((BN, 1), lambda i: (i, 0)),
            _full((H, H)), _full((H, H)),
            _full((1, H)), _full((1, H)), _full((2 * H, 1)), _full((1, 1)),
            _full((H, H)), _full((1, H)), _full((H, H)), _full((1, H)),
            _full((H, 1)), _full((1, 1)),
        ],
        out_specs=[
            pl.BlockSpec((BN, H), lambda i: (i, 0)),
            pl.BlockSpec((BN, H), lambda i: (i, 0)),
            pl.BlockSpec((BN, 1), lambda i: (i, 0)),
            pl.BlockSpec((BN, 1), lambda i: (i, 0)),
            pl.BlockSpec((BN, H), lambda i: (i, 0)),
        ],
        out_shape=[
            jax.ShapeDtypeStruct((N, H), _F32),
            jax.ShapeDtypeStruct((N, H), _F32),
            jax.ShapeDtypeStruct((N, 1), _F32),
            jax.ShapeDtypeStruct((N, 1), _F32),
            jax.ShapeDtypeStruct((N, H), _F32),
        ],
    )(Up, XW, degT, t2, Wgo, Wgt, bgo, bgt, Wa, ba, d1W, d1b, d2W, d2b,
      d3W, d3b)


def _mlp3_p(v, w1, b1, w2, b2, w3, b3):
    v = _lrelu(jnp.dot(v, w1[...], preferred_element_type=_F32) + b1[...])
    v = _lrelu(jnp.dot(v, w2[...], preferred_element_type=_F32) + b2[...])
    return jnp.dot(v, w3[...], preferred_element_type=_F32) + b3[...]


def _final_body(hraw_ref, den_ref, ro_ref, t_ref, weo_ref, weh_ref, be_ref,
                p1aW, p1ab, p1bW, p1bb, p1cW, p1cb,
                p0aW, p0ab, p0bW, p0bb, p0cW, p0cb,
                z2_ref, pred_ref):
    den = den_ref[:, 0:1] + den_ref[:, 1:2] + 1e-9
    h = (hraw_ref[0] + hraw_ref[1]) / den
    z2 = (jnp.dot(ro_ref[...], weo_ref[...], preferred_element_type=_F32)
          + jnp.dot(h, weh_ref[...], preferred_element_type=_F32)
          + be_ref[...])
    z2_ref[...] = z2
    p1 = _mlp3_p(z2, p1aW, p1ab, p1bW, p1bb, p1cW, p1cb)
    p0 = _mlp3_p(z2, p0aW, p0ab, p0bW, p0bb, p0cW, p0cb)
    pred_ref[...] = jnp.where(t_ref[...] > 0, p1, p0)


def _final_tc(hrawp, denT, ro, t2, Weo, Weh, be,
              p1aW, p1ab, p1bW, p1bb, p1cW, p1cb,
              p0aW, p0ab, p0bW, p0bb, p0cW, p0cb):
    return pl.pallas_call(
        _final_body,
        grid=(GRID,),
        in_specs=[
            pl.BlockSpec((NC, BN, H), lambda i: (0, i, 0)),
            pl.BlockSpec((BN, 2), lambda i: (i, 0)),
            pl.BlockSpec((BN, H), lambda i: (i, 0)),
            pl.BlockSpec((BN, 1), lambda i: (i, 0)),
            _full((H, H)), _full((H, H)), _full((1, H)),
            _full((H, H)), _full((1, H)), _full((H, H)), _full((1, H)),
            _full((H, 1)), _full((1, 1)),
            _full((H, H)), _full((1, H)), _full((H, H)), _full((1, H)),
            _full((H, 1)), _full((1, 1)),
        ],
        out_specs=[pl.BlockSpec((BN, H), lambda i: (i, 0)),
                   pl.BlockSpec((BN, 1), lambda i: (i, 0))],
        out_shape=[jax.ShapeDtypeStruct((N, H), _F32),
                   jax.ShapeDtypeStruct((N, 1), _F32)],
    )(hrawp, denT, ro, t2, Weo, Weh, be,
      p1aW, p1ab, p1bW, p1bb, p1cW, p1cb,
      p0aW, p0ab, p0bW, p0bb, p0cW, p0cb)


@jax.jit
def kernel(x, t, z, edge_index, Wgo, bgo, Wgt, bgt, Wa, ba, We, be,
           d1W, d1b, d2W, d2b, d3W, d3b, p1aW, p1ab, p1bW, p1bb, p1cW, p1cb,
           p0aW, p0ab, p0bW, p0bb, p0cW, p0cb):
    npad = EP - E
    ar = lax.iota(jnp.int32, npad)
    src2 = jnp.concatenate([edge_index[0], ar % 128]).reshape(NCHP, CH)
    dst2 = jnp.concatenate([edge_index[1], N + (ar % NPAD)]).reshape(NCHP, CH)
    t2 = t[:, None]
    zeros2 = jnp.zeros((N + NPAD, H), _F32)
    zrow = jnp.zeros((NP,), _F32)
    Wcat = jnp.concatenate([Wgo, Wgt], axis=1)

    deg0, deg1 = _deg_sc(dst2, zrow)
    degT = jnp.stack([deg0[:N], deg1[:N]], axis=1)
    XW, xd = _xw_tc(x, Wcat, degT)
    Up = _agg_sc(xd, src2, dst2, zeros2)
    ro, rt, pt, sm2, q = _gcnep_tc(
        Up, XW, degT, t2, Wgo, Wgt, bgo[None, :], bgt[None, :], Wa,
        ba[None, :], d1W, d1b[None, :], d2W, d2b[None, :], d3W, d3b[None, :])
    hrawp, den0, den1 = _att_sc(q, sm2[:, 0], src2, dst2, zeros2, zrow)
    denT = jnp.stack([den0[:N], den1[:N]], axis=1)
    z2, pred = _final_tc(
        hrawp, denT, ro, t2, We[:H], We[H:], be[None, :],
        p1aW, p1ab[None, :], p1bW, p1bb[None, :], p1cW, p1cb[None, :],
        p0aW, p0ab[None, :], p0bW, p0bb[None, :], p0cW, p0cb[None, :])
    return (pt, pred, z2)
